# baseline (speedup 1.0000x reference)
import jax
import jax.numpy as jnp
from jax import lax
from jax.experimental import pallas as pl
from jax.experimental.pallas import tpu as pltpu
from jax.experimental.pallas import tpu_sc as plsc

N = 10000
E = 320000
D = 128

NC = 2
NS = 16
NW = NC * NS

NP = 10240
ROWS_PT = NP // NS
DEG_PT = NP // NS

CH = 128
NCH = 2560
PE = NCH * CH
CPT = NCH // NW

_mesh = plsc.VectorSubcoreMesh(
    core_axis_name="c", subcore_axis_name="s", num_cores=NC, num_subcores=NS)


def _deg_body(dst2_hbm, d0_hbm, d1_hbm, deg_sp, zv, ones_v, didx_all):
    cid = lax.axis_index("c")
    sid = lax.axis_index("s")
    wid = cid * NS + sid

    def zinit(i, carry):
        zv[pl.ds(i * 16, 16)] = jnp.zeros((16,), jnp.float32)
        return carry

    lax.fori_loop(0, DEG_PT // 16, zinit, None)

    def oinit(i, carry):
        ones_v[pl.ds(i * 16, 16)] = jnp.ones((16,), jnp.float32)
        return carry

    lax.fori_loop(0, CH // 16, oinit, None)

    sl = pl.ds(sid * DEG_PT, DEG_PT)
    pltpu.sync_copy(zv, deg_sp.at[sl])
    plsc.subcore_barrier()

    base_e = wid * CPT * CH

    def body(c, carry):
        b = pl.multiple_of(base_e + c * CH, CH)
        pltpu.sync_copy(dst2_hbm.at[pl.ds(b, CH)], didx_all)
        pltpu.sync_copy(ones_v, deg_sp.at[didx_all], add=True)
        return carry

    lax.fori_loop(0, CPT, body, None)
    plsc.subcore_barrier()

    @pl.when(cid == 0)
    def _():
        pltpu.sync_copy(deg_sp.at[sl], d0_hbm.at[sl])

    @pl.when(cid == 1)
    def _():
        pltpu.sync_copy(deg_sp.at[sl], d1_hbm.at[sl])


_deg_kernel = pl.kernel(
    _deg_body,
    out_type=(jax.ShapeDtypeStruct((NP,), jnp.float32),
              jax.ShapeDtypeStruct((NP,), jnp.float32)),
    mesh=_mesh,
    scratch_types=[
        pltpu.VMEM_SHARED((NP,), jnp.float32),
        pltpu.VMEM((DEG_PT,), jnp.float32),
        pltpu.VMEM((CH,), jnp.float32),
        pltpu.VMEM((CH,), jnp.int32),
    ],
)


def _scat_body(src_hbm, dst_hbm, y_hbm, z_hbm, o0_hbm, o1_hbm,
               acc_sp, rA, siA, diA, gsA):
    cid = lax.axis_index("c")
    sid = lax.axis_index("s")
    wid = cid * NS + sid
    base_e = wid * CPT * CH

    rsl = pl.ds(sid * ROWS_PT, ROWS_PT)
    pltpu.sync_copy(z_hbm.at[rsl], acc_sp.at[rsl])
    plsc.subcore_barrier()

    def body(c, carry):
        b = pl.multiple_of(base_e + c * CH, CH)
        pltpu.sync_copy(src_hbm.at[pl.ds(b, CH)], siA)
        pltpu.sync_copy(dst_hbm.at[pl.ds(b, CH)], diA)
        pltpu.async_copy(y_hbm.at[siA], rA, gsA).wait()
        pltpu.sync_copy(rA, acc_sp.at[diA], add=True)
        return carry

    lax.fori_loop(0, CPT, body, None)
    plsc.subcore_barrier()

    @pl.when(cid == 0)
    def _():
        pltpu.sync_copy(acc_sp.at[rsl], o0_hbm.at[rsl])

    @pl.when(cid == 1)
    def _():
        pltpu.sync_copy(acc_sp.at[rsl], o1_hbm.at[rsl])


_scat_kernel = pl.kernel(
    _scat_body,
    out_type=(jax.ShapeDtypeStruct((NP, D), jnp.float32),
              jax.ShapeDtypeStruct((NP, D), jnp.float32)),
    mesh=_mesh,
    scratch_types=[
        pltpu.VMEM_SHARED((NP, D), jnp.float32),
        pltpu.VMEM((CH, D), jnp.float32),
        pltpu.VMEM((CH,), jnp.int32),
        pltpu.VMEM((CH,), jnp.int32),
        pltpu.SemaphoreType.DMA,
    ],
)

R1 = 160
G1 = NP // R1
R3 = 200
G3 = N // R3


def _tc1_body(x_ref, w_ref, d0_ref, d1_ref, dinv_ref, y1_ref):
    dv = lax.rsqrt(d0_ref[...] + d1_ref[...] + 1.0)
    dinv_ref[...] = dv
    y1_ref[...] = jnp.dot(
        x_ref[...], w_ref[...], preferred_element_type=jnp.float32) * dv


def _tc2_body(s0_ref, s1_ref, y1_ref, dv_ref, b1_ref, w2_ref, y2_ref):
    dv = dv_ref[...]
    h = jnp.maximum(dv * (s0_ref[...] + s1_ref[...] + y1_ref[...]) + b1_ref[...],
                    0.0)
    y2_ref[...] = jnp.dot(
        h, w2_ref[...], preferred_element_type=jnp.float32) * dv


def _tc3_body(s0_ref, s1_ref, y2_ref, dv_ref, b2_ref, out_ref):
    out_ref[...] = dv_ref[...] * (s0_ref[...] + s1_ref[...] + y2_ref[...]) \
        + b2_ref[...]


def _row_spec(r):
    return pl.BlockSpec((r, D), lambda i: (i, 0))


def _col_spec(r):
    return pl.BlockSpec((r, 1), lambda i: (i, 0))


_w_spec = pl.BlockSpec((D, D), lambda i: (0, 0))
_b_spec = pl.BlockSpec((1, D), lambda i: (0, 0))

_tc1 = pl.pallas_call(
    _tc1_body,
    grid=(G1,),
    in_specs=[_row_spec(R1), _w_spec, _col_spec(R1), _col_spec(R1)],
    out_specs=[_col_spec(R1), _row_spec(R1)],
    out_shape=(jax.ShapeDtypeStruct((NP, 1), jnp.float32),
               jax.ShapeDtypeStruct((NP, D), jnp.float32)),
)

_tc2 = pl.pallas_call(
    _tc2_body,
    grid=(G1,),
    in_specs=[_row_spec(R1), _row_spec(R1), _row_spec(R1), _col_spec(R1),
              _b_spec, _w_spec],
    out_specs=_row_spec(R1),
    out_shape=jax.ShapeDtypeStruct((NP, D), jnp.float32),
)

_tc3 = pl.pallas_call(
    _tc3_body,
    grid=(G3,),
    in_specs=[_row_spec(R3), _row_spec(R3), _row_spec(R3), _col_spec(R3),
              _b_spec],
    out_specs=_row_spec(R3),
    out_shape=jax.ShapeDtypeStruct((N, D), jnp.float32),
)


def kernel(x, edge_index, W1, b1, W2, b2):
    src = edge_index[0]
    dst = edge_index[1]
    pad_e = PE - E
    ppt = pad_e // NW
    pad_dst = jnp.broadcast_to(
        jnp.arange(N, N + ppt, dtype=jnp.int32), (NW, ppt))
    pad_src = jnp.full((NW, ppt), N, dtype=jnp.int32)
    srcp = jnp.concatenate(
        [src.reshape(NW, E // NW), pad_src], axis=1).reshape(NCH, CH)
    dstp = jnp.concatenate(
        [dst.reshape(NW, E // NW), pad_dst], axis=1).reshape(NCH, CH)
    xp = jnp.concatenate([x, jnp.zeros((NP - N, D), jnp.float32)])
    zeros = jnp.zeros((NP, D), jnp.float32)

    srcf = srcp.reshape(-1)
    dstf = dstp.reshape(-1)
    d0, d1 = _deg_kernel(dstf)
    d0 = d0.reshape(NP, 1)
    d1 = d1.reshape(NP, 1)

    dinv, y1 = _tc1(xp, W1, d0, d1)
    s10, s11 = _scat_kernel(srcf, dstf, y1, zeros)
    y2 = _tc2(s10, s11, y1, dinv, b1.reshape(1, D), W2)
    s20, s21 = _scat_kernel(srcf, dstf, y2, zeros)
    out = _tc3(s20, s21, y2, dinv, b2.reshape(1, D))
    return out


# speedup vs baseline: 1.7839x; 1.7839x over previous
import jax
import jax.numpy as jnp
from jax import lax
from jax.experimental import pallas as pl
from jax.experimental.pallas import tpu as pltpu
from jax.experimental.pallas import tpu_sc as plsc

N = 10000
E = 320000
D = 128

NC = 2
NS = 16
NW = NC * NS

NP = 10240
ROWS_PT = NP // NS
DEG_PT = NP // NS

CH = 128
NCH = 2560
PE = NCH * CH
CPT = NCH // NW

_mesh = plsc.VectorSubcoreMesh(
    core_axis_name="c", subcore_axis_name="s", num_cores=NC, num_subcores=NS)


def _deg_body(dst2_hbm, d0_hbm, d1_hbm, deg_sp, zv, ones_v, didx_all):
    cid = lax.axis_index("c")
    sid = lax.axis_index("s")
    wid = cid * NS + sid

    def zinit(i, carry):
        zv[pl.ds(i * 16, 16)] = jnp.zeros((16,), jnp.float32)
        return carry

    lax.fori_loop(0, DEG_PT // 16, zinit, None)

    def oinit(i, carry):
        ones_v[pl.ds(i * 16, 16)] = jnp.ones((16,), jnp.float32)
        return carry

    lax.fori_loop(0, CH // 16, oinit, None)

    sl = pl.ds(sid * DEG_PT, DEG_PT)
    pltpu.sync_copy(zv, deg_sp.at[sl])
    plsc.subcore_barrier()

    base_e = wid * CPT * CH

    def body(c, carry):
        b = pl.multiple_of(base_e + c * CH, CH)
        pltpu.sync_copy(dst2_hbm.at[pl.ds(b, CH)], didx_all)
        pltpu.sync_copy(ones_v, deg_sp.at[didx_all], add=True)
        return carry

    lax.fori_loop(0, CPT, body, None)
    plsc.subcore_barrier()

    @pl.when(cid == 0)
    def _():
        pltpu.sync_copy(deg_sp.at[sl], d0_hbm.at[sl])

    @pl.when(cid == 1)
    def _():
        pltpu.sync_copy(deg_sp.at[sl], d1_hbm.at[sl])


_deg_kernel = pl.kernel(
    _deg_body,
    out_type=(jax.ShapeDtypeStruct((NP,), jnp.float32),
              jax.ShapeDtypeStruct((NP,), jnp.float32)),
    mesh=_mesh,
    scratch_types=[
        pltpu.VMEM_SHARED((NP,), jnp.float32),
        pltpu.VMEM((DEG_PT,), jnp.float32),
        pltpu.VMEM((CH,), jnp.float32),
        pltpu.VMEM((CH,), jnp.int32),
    ],
)


def _scat_body(src_hbm, dst_hbm, y_hbm, z_hbm, o0_hbm, o1_hbm,
               acc_sp, rA, siA, diA, gsA):
    cid = lax.axis_index("c")
    sid = lax.axis_index("s")
    wid = cid * NS + sid
    base_e = wid * CPT * CH

    rsl = pl.ds(sid * ROWS_PT, ROWS_PT)
    pltpu.sync_copy(z_hbm.at[rsl], acc_sp.at[rsl])
    plsc.subcore_barrier()

    def body(c, carry):
        b = pl.multiple_of(base_e + c * CH, CH)
        pltpu.sync_copy(src_hbm.at[pl.ds(b, CH)], siA)
        pltpu.sync_copy(dst_hbm.at[pl.ds(b, CH)], diA)
        pltpu.async_copy(y_hbm.at[siA], rA, gsA).wait()
        pltpu.sync_copy(rA, acc_sp.at[diA], add=True)
        return carry

    lax.fori_loop(0, CPT, body, None)
    plsc.subcore_barrier()

    @pl.when(cid == 0)
    def _():
        pltpu.sync_copy(acc_sp.at[rsl], o0_hbm.at[rsl])

    @pl.when(cid == 1)
    def _():
        pltpu.sync_copy(acc_sp.at[rsl], o1_hbm.at[rsl])


_scat_kernel = pl.kernel(
    _scat_body,
    out_type=(jax.ShapeDtypeStruct((NP, D), jnp.float32),
              jax.ShapeDtypeStruct((NP, D), jnp.float32)),
    mesh=_mesh,
    scratch_types=[
        pltpu.VMEM_SHARED((NP, D), jnp.float32),
        pltpu.VMEM((CH, D), jnp.float32),
        pltpu.VMEM((CH,), jnp.int32),
        pltpu.VMEM((CH,), jnp.int32),
        pltpu.SemaphoreType.DMA,
    ],
)

R1 = 160
G1 = NP // R1
R3 = 200
G3 = N // R3


def _tc1_body(x_ref, w_ref, d0_ref, d1_ref, dinv_ref, y1_ref):
    dv = lax.rsqrt(d0_ref[...] + d1_ref[...] + 1.0)
    dinv_ref[...] = dv
    y1_ref[...] = jnp.dot(
        x_ref[...], w_ref[...], preferred_element_type=jnp.float32) * dv


def _tc2_body(s0_ref, s1_ref, y1_ref, dv_ref, b1_ref, w2_ref, y2_ref):
    dv = dv_ref[...]
    h = jnp.maximum(dv * (s0_ref[...] + s1_ref[...] + y1_ref[...]) + b1_ref[...],
                    0.0)
    y2_ref[...] = jnp.dot(
        h, w2_ref[...], preferred_element_type=jnp.float32) * dv


def _tc3_body(s0_ref, s1_ref, y2_ref, dv_ref, b2_ref, out_ref):
    out_ref[...] = dv_ref[...] * (s0_ref[...] + s1_ref[...] + y2_ref[...]) \
        + b2_ref[...]


def _row_spec(r):
    return pl.BlockSpec((r, D), lambda i: (i, 0))


def _col_spec(r):
    return pl.BlockSpec((r, 1), lambda i: (i, 0))


_w_spec = pl.BlockSpec((D, D), lambda i: (0, 0))
_b_spec = pl.BlockSpec((1, D), lambda i: (0, 0))

_tc1 = pl.pallas_call(
    _tc1_body,
    grid=(G1,),
    in_specs=[_row_spec(R1), _w_spec, _col_spec(R1), _col_spec(R1)],
    out_specs=[_col_spec(R1), _row_spec(R1)],
    out_shape=(jax.ShapeDtypeStruct((NP, 1), jnp.float32),
               jax.ShapeDtypeStruct((NP, D), jnp.float32)),
)

_tc2 = pl.pallas_call(
    _tc2_body,
    grid=(G1,),
    in_specs=[_row_spec(R1), _row_spec(R1), _row_spec(R1), _col_spec(R1),
              _b_spec, _w_spec],
    out_specs=_row_spec(R1),
    out_shape=jax.ShapeDtypeStruct((NP, D), jnp.float32),
)

_tc3 = pl.pallas_call(
    _tc3_body,
    grid=(G3,),
    in_specs=[_row_spec(R3), _row_spec(R3), _row_spec(R3), _col_spec(R3),
              _b_spec],
    out_specs=_row_spec(R3),
    out_shape=jax.ShapeDtypeStruct((N, D), jnp.float32),
)


def kernel(x, edge_index, W1, b1, W2, b2):
    src = edge_index[0]
    dst = edge_index[1]
    pad_e = PE - E
    ppt = pad_e // NW
    pad_dst = jnp.broadcast_to(
        jnp.arange(N, N + ppt, dtype=jnp.int32), (NW, ppt))
    pad_src = pad_dst
    srcp = jnp.concatenate(
        [src.reshape(NW, E // NW), pad_src], axis=1).reshape(NCH, CH)
    dstp = jnp.concatenate(
        [dst.reshape(NW, E // NW), pad_dst], axis=1).reshape(NCH, CH)
    xp = jnp.concatenate([x, jnp.zeros((NP - N, D), jnp.float32)])
    zeros = jnp.zeros((NP, D), jnp.float32)

    srcf = srcp.reshape(-1)
    dstf = dstp.reshape(-1)
    d0, d1 = _deg_kernel(dstf)
    d0 = d0.reshape(NP, 1)
    d1 = d1.reshape(NP, 1)

    dinv, y1 = _tc1(xp, W1, d0, d1)
    s10, s11 = _scat_kernel(srcf, dstf, y1, zeros)
    y2 = _tc2(s10, s11, y1, dinv, b1.reshape(1, D), W2)
    s20, s21 = _scat_kernel(srcf, dstf, y2, zeros)
    out = _tc3(s20, s21, y2, dinv, b2.reshape(1, D))
    return out


# speedup vs baseline: 2.7679x; 1.5516x over previous
import jax
import jax.numpy as jnp
from jax import lax
from jax.experimental import pallas as pl
from jax.experimental.pallas import tpu as pltpu
from jax.experimental.pallas import tpu_sc as plsc

N = 10000
E = 320000
D = 128

NC = 2
NS = 16
NW = NC * NS

NP = 10240
ROWS_PT = NP // NS
DEG_PT = NP // NS

CH = 128
NCH = 2560
PE = NCH * CH
CPT = NCH // NW

_mesh = plsc.VectorSubcoreMesh(
    core_axis_name="c", subcore_axis_name="s", num_cores=NC, num_subcores=NS)


def _deg_body(dst2_hbm, d0_hbm, d1_hbm, deg_sp, zv, ones_v, didx_all):
    cid = lax.axis_index("c")
    sid = lax.axis_index("s")
    wid = cid * NS + sid

    def zinit(i, carry):
        zv[pl.ds(i * 16, 16)] = jnp.zeros((16,), jnp.float32)
        return carry

    lax.fori_loop(0, DEG_PT // 16, zinit, None)

    def oinit(i, carry):
        ones_v[pl.ds(i * 16, 16)] = jnp.ones((16,), jnp.float32)
        return carry

    lax.fori_loop(0, CH // 16, oinit, None)

    sl = pl.ds(sid * DEG_PT, DEG_PT)
    pltpu.sync_copy(zv, deg_sp.at[sl])
    plsc.subcore_barrier()

    base_e = wid * CPT * CH

    def body(c, carry):
        b = pl.multiple_of(base_e + c * CH, CH)
        pltpu.sync_copy(dst2_hbm.at[pl.ds(b, CH)], didx_all)
        pltpu.sync_copy(ones_v, deg_sp.at[didx_all], add=True)
        return carry

    lax.fori_loop(0, CPT, body, None)
    plsc.subcore_barrier()

    @pl.when(cid == 0)
    def _():
        pltpu.sync_copy(deg_sp.at[sl], d0_hbm.at[sl])

    @pl.when(cid == 1)
    def _():
        pltpu.sync_copy(deg_sp.at[sl], d1_hbm.at[sl])


_deg_kernel = pl.kernel(
    _deg_body,
    out_type=(jax.ShapeDtypeStruct((NP,), jnp.float32),
              jax.ShapeDtypeStruct((NP,), jnp.float32)),
    mesh=_mesh,
    scratch_types=[
        pltpu.VMEM_SHARED((NP,), jnp.float32),
        pltpu.VMEM((DEG_PT,), jnp.float32),
        pltpu.VMEM((CH,), jnp.float32),
        pltpu.VMEM((CH,), jnp.int32),
    ],
)


def _scat_body(src_hbm, dst_hbm, y_hbm, z_hbm, o0_hbm, o1_hbm,
               acc_sp, rA, rB, siA, diA, siB, diB, gsA, gsB, isA, isB):
    cid = lax.axis_index("c")
    sid = lax.axis_index("s")
    wid = cid * NS + sid
    base_e = wid * CPT * CH

    rsl = pl.ds(sid * ROWS_PT, ROWS_PT)
    pltpu.sync_copy(z_hbm.at[rsl], acc_sp.at[rsl])
    plsc.subcore_barrier()

    def idx_load(c, si, di, isem):
        b = pl.multiple_of(base_e + c * CH, CH)
        pltpu.async_copy(src_hbm.at[pl.ds(b, CH)], si, isem)
        pltpu.async_copy(dst_hbm.at[pl.ds(b, CH)], di, isem)

    def idx_wait(c, si, di, isem):
        b = pl.multiple_of(base_e + c * CH, CH)
        pltpu.make_async_copy(src_hbm.at[pl.ds(b, CH)], si, isem).wait()
        pltpu.make_async_copy(dst_hbm.at[pl.ds(b, CH)], di, isem).wait()

    def gather(si, rows, gsem):
        pltpu.async_copy(y_hbm.at[si], rows, gsem)

    def gwait(si, rows, gsem):
        pltpu.make_async_copy(y_hbm.at[si], rows, gsem).wait()

    idx_load(0, siA, diA, isA)
    idx_load(1, siB, diB, isB)
    idx_wait(0, siA, diA, isA)
    gather(siA, rA, gsA)

    def step(c, cur, nxt):
        r_c, si_c, di_c, gs_c, is_c = cur
        r_n, si_n, di_n, gs_n, is_n = nxt

        @pl.when(c + 1 < CPT)
        def _():
            idx_wait(c + 1, si_n, di_n, is_n)
            gather(si_n, r_n, gs_n)

        gwait(si_c, r_c, gs_c)
        pltpu.sync_copy(r_c, acc_sp.at[di_c], add=True)

        @pl.when(c + 2 < CPT)
        def _():
            idx_load(c + 2, si_c, di_c, is_c)

    bufA = (rA, siA, diA, gsA, isA)
    bufB = (rB, siB, diB, gsB, isB)

    def body(t, carry):
        step(2 * t, bufA, bufB)
        step(2 * t + 1, bufB, bufA)
        return carry

    lax.fori_loop(0, CPT // 2, body, None)
    plsc.subcore_barrier()

    @pl.when(cid == 0)
    def _():
        pltpu.sync_copy(acc_sp.at[rsl], o0_hbm.at[rsl])

    @pl.when(cid == 1)
    def _():
        pltpu.sync_copy(acc_sp.at[rsl], o1_hbm.at[rsl])


_scat_kernel = pl.kernel(
    _scat_body,
    out_type=(jax.ShapeDtypeStruct((NP, D), jnp.float32),
              jax.ShapeDtypeStruct((NP, D), jnp.float32)),
    mesh=_mesh,
    scratch_types=[
        pltpu.VMEM_SHARED((NP, D), jnp.float32),
        pltpu.VMEM((CH, D), jnp.float32),
        pltpu.VMEM((CH, D), jnp.float32),
        pltpu.VMEM((CH,), jnp.int32),
        pltpu.VMEM((CH,), jnp.int32),
        pltpu.VMEM((CH,), jnp.int32),
        pltpu.VMEM((CH,), jnp.int32),
        pltpu.SemaphoreType.DMA,
        pltpu.SemaphoreType.DMA,
        pltpu.SemaphoreType.DMA,
        pltpu.SemaphoreType.DMA,
    ],
)

R1 = 160
G1 = NP // R1
R3 = 200
G3 = N // R3


def _tc1_body(x_ref, w_ref, d0_ref, d1_ref, dinv_ref, y1_ref):
    dv = lax.rsqrt(d0_ref[...] + d1_ref[...] + 1.0)
    dinv_ref[...] = dv
    y1_ref[...] = jnp.dot(
        x_ref[...], w_ref[...], preferred_element_type=jnp.float32) * dv


def _tc2_body(s0_ref, s1_ref, y1_ref, dv_ref, b1_ref, w2_ref, y2_ref):
    dv = dv_ref[...]
    h = jnp.maximum(dv * (s0_ref[...] + s1_ref[...] + y1_ref[...]) + b1_ref[...],
                    0.0)
    y2_ref[...] = jnp.dot(
        h, w2_ref[...], preferred_element_type=jnp.float32) * dv


def _tc3_body(s0_ref, s1_ref, y2_ref, dv_ref, b2_ref, out_ref):
    out_ref[...] = dv_ref[...] * (s0_ref[...] + s1_ref[...] + y2_ref[...]) \
        + b2_ref[...]


def _row_spec(r):
    return pl.BlockSpec((r, D), lambda i: (i, 0))


def _col_spec(r):
    return pl.BlockSpec((r, 1), lambda i: (i, 0))


_w_spec = pl.BlockSpec((D, D), lambda i: (0, 0))
_b_spec = pl.BlockSpec((1, D), lambda i: (0, 0))

_tc1 = pl.pallas_call(
    _tc1_body,
    grid=(G1,),
    in_specs=[_row_spec(R1), _w_spec, _col_spec(R1), _col_spec(R1)],
    out_specs=[_col_spec(R1), _row_spec(R1)],
    out_shape=(jax.ShapeDtypeStruct((NP, 1), jnp.float32),
               jax.ShapeDtypeStruct((NP, D), jnp.float32)),
)

_tc2 = pl.pallas_call(
    _tc2_body,
    grid=(G1,),
    in_specs=[_row_spec(R1), _row_spec(R1), _row_spec(R1), _col_spec(R1),
              _b_spec, _w_spec],
    out_specs=_row_spec(R1),
    out_shape=jax.ShapeDtypeStruct((NP, D), jnp.float32),
)

_tc3 = pl.pallas_call(
    _tc3_body,
    grid=(G3,),
    in_specs=[_row_spec(R3), _row_spec(R3), _row_spec(R3), _col_spec(R3),
              _b_spec],
    out_specs=_row_spec(R3),
    out_shape=jax.ShapeDtypeStruct((N, D), jnp.float32),
)


def kernel(x, edge_index, W1, b1, W2, b2):
    src = edge_index[0]
    dst = edge_index[1]
    pad_e = PE - E
    ppt = pad_e // NW
    pad_dst = jnp.broadcast_to(
        jnp.arange(N, N + ppt, dtype=jnp.int32), (NW, ppt))
    pad_src = pad_dst
    srcp = jnp.concatenate(
        [src.reshape(NW, E // NW), pad_src], axis=1).reshape(NCH, CH)
    dstp = jnp.concatenate(
        [dst.reshape(NW, E // NW), pad_dst], axis=1).reshape(NCH, CH)
    xp = jnp.concatenate([x, jnp.zeros((NP - N, D), jnp.float32)])
    zeros = jnp.zeros((NP, D), jnp.float32)

    srcf = srcp.reshape(-1)
    dstf = dstp.reshape(-1)
    d0, d1 = _deg_kernel(dstf)
    d0 = d0.reshape(NP, 1)
    d1 = d1.reshape(NP, 1)

    dinv, y1 = _tc1(xp, W1, d0, d1)
    s10, s11 = _scat_kernel(srcf, dstf, y1, zeros)
    y2 = _tc2(s10, s11, y1, dinv, b1.reshape(1, D), W2)
    s20, s21 = _scat_kernel(srcf, dstf, y2, zeros)
    out = _tc3(s20, s21, y2, dinv, b2.reshape(1, D))
    return out


# speedup vs baseline: 3.0403x; 1.0984x over previous
import jax
import jax.numpy as jnp
from jax import lax
from jax.experimental import pallas as pl
from jax.experimental.pallas import tpu as pltpu
from jax.experimental.pallas import tpu_sc as plsc

N = 10000
E = 320000
D = 128

NC = 2
NS = 16
NW = NC * NS

NP = 10240
ROWS_PT = NP // NS
DEG_PT = NP // NS

CH = 128
NCH = 2560
PE = NCH * CH
CPT = NCH // NW
DGRP = 8
NDG = CPT // DGRP

_mesh = plsc.VectorSubcoreMesh(
    core_axis_name="c", subcore_axis_name="s", num_cores=NC, num_subcores=NS)


def _deg_body(dst2_hbm, d0_hbm, d1_hbm, deg_sp, zv, ones_v, didx_all, dsem):
    cid = lax.axis_index("c")
    sid = lax.axis_index("s")
    wid = cid * NS + sid

    def zinit(i, carry):
        zv[pl.ds(i * 16, 16)] = jnp.zeros((16,), jnp.float32)
        return carry

    lax.fori_loop(0, DEG_PT // 16, zinit, None)

    def oinit(i, carry):
        ones_v[pl.ds(i * 16, 16)] = jnp.ones((16,), jnp.float32)
        return carry

    lax.fori_loop(0, CH // 16, oinit, None)

    sl = pl.ds(sid * DEG_PT, DEG_PT)
    pltpu.sync_copy(zv, deg_sp.at[sl])
    pltpu.sync_copy(dst2_hbm.at[pl.ds(wid * CPT, CPT)], didx_all)
    plsc.subcore_barrier()

    def fire(g):
        def k(i, carry):
            pltpu.async_copy(
                ones_v, deg_sp.at[didx_all.at[g * DGRP + i]], dsem, add=True)
            return carry
        lax.fori_loop(0, DGRP, k, None)

    def drain(g):
        def k(i, carry):
            pltpu.make_async_copy(
                ones_v, deg_sp.at[didx_all.at[g * DGRP + i]], dsem).wait()
            return carry
        lax.fori_loop(0, DGRP, k, None)

    fire(0)

    def body(g, carry):
        @pl.when(g + 1 < NDG)
        def _():
            fire(g + 1)
        drain(g)
        return carry

    lax.fori_loop(0, NDG, body, None)
    plsc.subcore_barrier()

    @pl.when(cid == 0)
    def _():
        pltpu.sync_copy(deg_sp.at[sl], d0_hbm.at[sl])

    @pl.when(cid == 1)
    def _():
        pltpu.sync_copy(deg_sp.at[sl], d1_hbm.at[sl])


_deg_kernel = pl.kernel(
    _deg_body,
    out_type=(jax.ShapeDtypeStruct((NP,), jnp.float32),
              jax.ShapeDtypeStruct((NP,), jnp.float32)),
    mesh=_mesh,
    scratch_types=[
        pltpu.VMEM_SHARED((NP,), jnp.float32),
        pltpu.VMEM((DEG_PT,), jnp.float32),
        pltpu.VMEM((CH,), jnp.float32),
        pltpu.VMEM((CPT, CH), jnp.int32),
        pltpu.SemaphoreType.DMA,
    ],
)


def _scat_body(src_hbm, dst_hbm, y_hbm, z_hbm, o0_hbm, o1_hbm,
               acc_sp, rA, rB, siA, diA, siB, diB, gsA, gsB, isA, isB):
    cid = lax.axis_index("c")
    sid = lax.axis_index("s")
    wid = cid * NS + sid
    base_e = wid * CPT * CH

    rsl = pl.ds(sid * ROWS_PT, ROWS_PT)
    pltpu.sync_copy(z_hbm.at[rsl], acc_sp.at[rsl])
    plsc.subcore_barrier()

    def idx_load(c, si, di, isem):
        b = pl.multiple_of(base_e + c * CH, CH)
        pltpu.async_copy(src_hbm.at[pl.ds(b, CH)], si, isem)
        pltpu.async_copy(dst_hbm.at[pl.ds(b, CH)], di, isem)

    def idx_wait(c, si, di, isem):
        b = pl.multiple_of(base_e + c * CH, CH)
        pltpu.make_async_copy(src_hbm.at[pl.ds(b, CH)], si, isem).wait()
        pltpu.make_async_copy(dst_hbm.at[pl.ds(b, CH)], di, isem).wait()

    def gather(si, rows, gsem):
        pltpu.async_copy(y_hbm.at[si], rows, gsem)

    def gwait(si, rows, gsem):
        pltpu.make_async_copy(y_hbm.at[si], rows, gsem).wait()

    idx_load(0, siA, diA, isA)
    idx_load(1, siB, diB, isB)
    idx_wait(0, siA, diA, isA)
    gather(siA, rA, gsA)

    def step(c, cur, nxt):
        r_c, si_c, di_c, gs_c, is_c = cur
        r_n, si_n, di_n, gs_n, is_n = nxt

        @pl.when(c + 1 < CPT)
        def _():
            idx_wait(c + 1, si_n, di_n, is_n)
            gather(si_n, r_n, gs_n)

        gwait(si_c, r_c, gs_c)
        pltpu.sync_copy(r_c, acc_sp.at[di_c], add=True)

        @pl.when(c + 2 < CPT)
        def _():
            idx_load(c + 2, si_c, di_c, is_c)

    bufA = (rA, siA, diA, gsA, isA)
    bufB = (rB, siB, diB, gsB, isB)

    def body(t, carry):
        step(2 * t, bufA, bufB)
        step(2 * t + 1, bufB, bufA)
        return carry

    lax.fori_loop(0, CPT // 2, body, None)
    plsc.subcore_barrier()

    @pl.when(cid == 0)
    def _():
        pltpu.sync_copy(acc_sp.at[rsl], o0_hbm.at[rsl])

    @pl.when(cid == 1)
    def _():
        pltpu.sync_copy(acc_sp.at[rsl], o1_hbm.at[rsl])


_scat_kernel = pl.kernel(
    _scat_body,
    out_type=(jax.ShapeDtypeStruct((NP, D), jnp.float32),
              jax.ShapeDtypeStruct((NP, D), jnp.float32)),
    mesh=_mesh,
    scratch_types=[
        pltpu.VMEM_SHARED((NP, D), jnp.float32),
        pltpu.VMEM((CH, D), jnp.float32),
        pltpu.VMEM((CH, D), jnp.float32),
        pltpu.VMEM((CH,), jnp.int32),
        pltpu.VMEM((CH,), jnp.int32),
        pltpu.VMEM((CH,), jnp.int32),
        pltpu.VMEM((CH,), jnp.int32),
        pltpu.SemaphoreType.DMA,
        pltpu.SemaphoreType.DMA,
        pltpu.SemaphoreType.DMA,
        pltpu.SemaphoreType.DMA,
    ],
)

R1 = 160
G1 = NP // R1
R3 = 200
G3 = N // R3


def _tc1_body(x_ref, w_ref, d0_ref, d1_ref, dinv_ref, y1_ref):
    dv = lax.rsqrt(d0_ref[...] + d1_ref[...] + 1.0)
    dinv_ref[...] = dv
    y1_ref[...] = jnp.dot(
        x_ref[...], w_ref[...], preferred_element_type=jnp.float32) * dv


def _tc2_body(s0_ref, s1_ref, y1_ref, dv_ref, b1_ref, w2_ref, y2_ref):
    dv = dv_ref[...]
    h = jnp.maximum(dv * (s0_ref[...] + s1_ref[...] + y1_ref[...]) + b1_ref[...],
                    0.0)
    y2_ref[...] = jnp.dot(
        h, w2_ref[...], preferred_element_type=jnp.float32) * dv


def _tc3_body(s0_ref, s1_ref, y2_ref, dv_ref, b2_ref, out_ref):
    out_ref[...] = dv_ref[...] * (s0_ref[...] + s1_ref[...] + y2_ref[...]) \
        + b2_ref[...]


def _row_spec(r):
    return pl.BlockSpec((r, D), lambda i: (i, 0))


def _col_spec(r):
    return pl.BlockSpec((r, 1), lambda i: (i, 0))


_w_spec = pl.BlockSpec((D, D), lambda i: (0, 0))
_b_spec = pl.BlockSpec((1, D), lambda i: (0, 0))

_tc1 = pl.pallas_call(
    _tc1_body,
    grid=(G1,),
    in_specs=[_row_spec(R1), _w_spec, _col_spec(R1), _col_spec(R1)],
    out_specs=[_col_spec(R1), _row_spec(R1)],
    out_shape=(jax.ShapeDtypeStruct((NP, 1), jnp.float32),
               jax.ShapeDtypeStruct((NP, D), jnp.float32)),
)

_tc2 = pl.pallas_call(
    _tc2_body,
    grid=(G1,),
    in_specs=[_row_spec(R1), _row_spec(R1), _row_spec(R1), _col_spec(R1),
              _b_spec, _w_spec],
    out_specs=_row_spec(R1),
    out_shape=jax.ShapeDtypeStruct((NP, D), jnp.float32),
)

_tc3 = pl.pallas_call(
    _tc3_body,
    grid=(G3,),
    in_specs=[_row_spec(R3), _row_spec(R3), _row_spec(R3), _col_spec(R3),
              _b_spec],
    out_specs=_row_spec(R3),
    out_shape=jax.ShapeDtypeStruct((N, D), jnp.float32),
)


def kernel(x, edge_index, W1, b1, W2, b2):
    src = edge_index[0]
    dst = edge_index[1]
    pad_e = PE - E
    ppt = pad_e // NW
    pad_dst = jnp.broadcast_to(
        jnp.arange(N, N + ppt, dtype=jnp.int32), (NW, ppt))
    pad_src = pad_dst
    srcp = jnp.concatenate(
        [src.reshape(NW, E // NW), pad_src], axis=1).reshape(NCH, CH)
    dstp = jnp.concatenate(
        [dst.reshape(NW, E // NW), pad_dst], axis=1).reshape(NCH, CH)
    xp = jnp.concatenate([x, jnp.zeros((NP - N, D), jnp.float32)])
    zeros = jnp.zeros((NP, D), jnp.float32)

    srcf = srcp.reshape(-1)
    dstf = dstp.reshape(-1)
    d0, d1 = _deg_kernel(dstp)
    d0 = d0.reshape(NP, 1)
    d1 = d1.reshape(NP, 1)

    dinv, y1 = _tc1(xp, W1, d0, d1)
    s10, s11 = _scat_kernel(srcf, dstf, y1, zeros)
    y2 = _tc2(s10, s11, y1, dinv, b1.reshape(1, D), W2)
    s20, s21 = _scat_kernel(srcf, dstf, y2, zeros)
    out = _tc3(s20, s21, y2, dinv, b2.reshape(1, D))
    return out
